# baseline (speedup 1.0000x reference)
import functools

import jax
import jax.numpy as jnp
from jax import lax
from jax.experimental import pallas as pl
from jax.experimental.pallas import tpu as pltpu
from jax.experimental.pallas import tpu_sc as plsc

N = 100000
E = 6400000
NPAD = 102400
ROWS2D = 800
PADC = NPAD - N
NCORE = 2
NSUB = 16
NW = NCORE * NSUB
EPW = E // NW
RPS = NPAD // NSUB
FN = jnp.float32


def _fill_1d(ref, n, val):
    def body(i, c):
        ref[pl.ds(i * 16, 16)] = jnp.full((16,), val, FN)
        return c
    lax.fori_loop(0, n // 16, body, 0)


def _zero_rows_2d(ref, n):
    def body(i, c):
        ref[i, :] = jnp.zeros((16,), FN)
        return c
    lax.fori_loop(0, n, body, 0)


def _deg_pass(dst):
    C = 2000
    CZ = 800
    mesh = plsc.VectorSubcoreMesh(core_axis_name="c", subcore_axis_name="s")

    @functools.partial(
        pl.kernel,
        out_type=jax.ShapeDtypeStruct((NCORE * NPAD,), FN),
        mesh=mesh,
        scratch_types=[
            pltpu.VMEM_SHARED((NPAD,), FN),
            pltpu.VMEM((C,), jnp.int32),
            pltpu.VMEM((C,), jnp.int32),
            pltpu.VMEM((C,), FN),
            pltpu.SemaphoreType.DMA,
            pltpu.SemaphoreType.DMA,
        ],
    )
    def k(dst_hbm, out_hbm, acc, idx0, idx1, ones, s0, s1):
        cid = lax.axis_index("c")
        sid = lax.axis_index("s")
        _fill_1d(ones, C, 0.0)

        def zbody(kk, c):
            pltpu.sync_copy(ones.at[pl.ds(0, CZ)],
                            acc.at[pl.ds(sid * RPS + kk * CZ, CZ)])
            return c
        lax.fori_loop(0, RPS // CZ, zbody, 0)
        _fill_1d(ones, C, 1.0)
        plsc.subcore_barrier()

        base0 = (cid * NSUB + sid) * EPW

        def ebody(t, c):
            ba = pl.multiple_of(base0 + (2 * t) * C, 8)
            bb = pl.multiple_of(base0 + (2 * t + 1) * C, 8)
            pltpu.sync_copy(dst_hbm.at[pl.ds(ba, C)], idx0)
            sa = pltpu.async_copy(ones, acc.at[idx0], s0, add=True)
            pltpu.sync_copy(dst_hbm.at[pl.ds(bb, C)], idx1)
            sb = pltpu.async_copy(ones, acc.at[idx1], s1, add=True)
            sa.wait()
            sb.wait()
            return c
        lax.fori_loop(0, EPW // C // 2, ebody, 0)
        plsc.subcore_barrier()

        def wbody(kk, c):
            r0 = sid * RPS + kk * CZ
            pltpu.sync_copy(acc.at[pl.ds(r0, CZ)], ones.at[pl.ds(0, CZ)])
            pltpu.sync_copy(ones.at[pl.ds(0, CZ)],
                            out_hbm.at[pl.ds(cid * NPAD + r0, CZ)])
            return c
        lax.fori_loop(0, RPS // CZ, wbody, 0)

    return k(dst)


def _scalar_pass(src, dst, table):
    C = 2000
    CZ = 800
    ST = 1600
    mesh = plsc.VectorSubcoreMesh(core_axis_name="c", subcore_axis_name="s")

    @functools.partial(
        pl.kernel,
        out_type=jax.ShapeDtypeStruct((NCORE * NPAD,), FN),
        mesh=mesh,
        scratch_types=[
            pltpu.VMEM_SHARED((NPAD,), FN),
            pltpu.VMEM_SHARED((NPAD,), FN),
            pltpu.VMEM((C,), jnp.int32),
            pltpu.VMEM((C,), jnp.int32),
            pltpu.VMEM((C,), jnp.int32),
            pltpu.VMEM((C,), jnp.int32),
            pltpu.VMEM((C,), jnp.int32),
            pltpu.VMEM((C,), jnp.int32),
            pltpu.VMEM((C,), jnp.int32),
            pltpu.VMEM((C,), jnp.int32),
            pltpu.VMEM((C,), FN),
            pltpu.VMEM((C,), FN),
            pltpu.SemaphoreType.DMA,
            pltpu.SemaphoreType.DMA,
            pltpu.SemaphoreType.DMA,
            pltpu.SemaphoreType.DMA,
            pltpu.SemaphoreType.DMA,
            pltpu.SemaphoreType.DMA,
        ],
    )
    def k(src_hbm, dst_hbm, tab_hbm, out_hbm, acc, tab, idxsA0, idxdA0,
          idxsA1, idxdA1, idxsB0, idxdB0, idxsB1, idxdB1, vals0, vals1,
          g0, g1, w0, w1, pA, pB):
        cid = lax.axis_index("c")
        sid = lax.axis_index("s")
        _fill_1d(vals0, C, 0.0)

        def zbody(kk, c):
            pltpu.sync_copy(vals0.at[pl.ds(0, CZ)],
                            acc.at[pl.ds(sid * RPS + kk * CZ, CZ)])
            return c
        lax.fori_loop(0, RPS // CZ, zbody, 0)

        def stage(q, c):
            r0 = sid * RPS + q * ST
            pltpu.sync_copy(tab_hbm.at[pl.ds(r0, ST)], vals1.at[pl.ds(0, ST)])
            pltpu.sync_copy(vals1.at[pl.ds(0, ST)], tab.at[pl.ds(r0, ST)])
            return c
        lax.fori_loop(0, RPS // ST, stage, 0)
        plsc.subcore_barrier()

        base0 = (cid * NSUB + sid) * EPW
        Q = EPW // C // 4

        def pre(bx, by, sA, dA, sB, dB, sem):
            pltpu.async_copy(src_hbm.at[pl.ds(bx, C)], sA, sem)
            pltpu.async_copy(dst_hbm.at[pl.ds(bx, C)], dA, sem)
            pltpu.async_copy(src_hbm.at[pl.ds(by, C)], sB, sem)
            pltpu.async_copy(dst_hbm.at[pl.ds(by, C)], dB, sem)

        def drain(bx, by, sA, dA, sB, dB, sem):
            pltpu.make_async_copy(src_hbm.at[pl.ds(bx, C)], sA, sem).wait()
            pltpu.make_async_copy(dst_hbm.at[pl.ds(bx, C)], dA, sem).wait()
            pltpu.make_async_copy(src_hbm.at[pl.ds(by, C)], sB, sem).wait()
            pltpu.make_async_copy(dst_hbm.at[pl.ds(by, C)], dB, sem).wait()

        pre(base0, base0 + C, idxsA0, idxdA0, idxsA1, idxdA1, pA)

        def ebody(q, c):
            b0 = pl.multiple_of(base0 + (4 * q) * C, 8)
            b1 = pl.multiple_of(base0 + (4 * q + 1) * C, 8)
            b2 = pl.multiple_of(base0 + (4 * q + 2) * C, 8)
            b3 = pl.multiple_of(base0 + (4 * q + 3) * C, 8)
            nxt = jnp.where(q + 1 < Q, base0 + (4 * q + 4) * C, base0)
            nxt = pl.multiple_of(nxt, 8)
            drain(b0, b1, idxsA0, idxdA0, idxsA1, idxdA1, pA)
            ga = pltpu.async_copy(tab.at[idxsA0], vals0, g0)
            gb = pltpu.async_copy(tab.at[idxsA1], vals1, g1)
            pre(b2, b3, idxsB0, idxdB0, idxsB1, idxdB1, pB)
            ga.wait()
            sa = pltpu.async_copy(vals0, acc.at[idxdA0], w0, add=True)
            gb.wait()
            sb = pltpu.async_copy(vals1, acc.at[idxdA1], w1, add=True)
            sa.wait()
            sb.wait()
            drain(b2, b3, idxsB0, idxdB0, idxsB1, idxdB1, pB)
            ga2 = pltpu.async_copy(tab.at[idxsB0], vals0, g0)
            gb2 = pltpu.async_copy(tab.at[idxsB1], vals1, g1)
            pre(nxt, nxt + C, idxsA0, idxdA0, idxsA1, idxdA1, pA)
            ga2.wait()
            sa2 = pltpu.async_copy(vals0, acc.at[idxdB0], w0, add=True)
            gb2.wait()
            sb2 = pltpu.async_copy(vals1, acc.at[idxdB1], w1, add=True)
            sa2.wait()
            sb2.wait()
            return c
        lax.fori_loop(0, Q, ebody, 0)
        drain(base0, base0 + C, idxsA0, idxdA0, idxsA1, idxdA1, pA)
        plsc.subcore_barrier()

        def wbody(kk, c):
            r0 = sid * RPS + kk * CZ
            pltpu.sync_copy(acc.at[pl.ds(r0, CZ)], vals0.at[pl.ds(0, CZ)])
            pltpu.sync_copy(vals0.at[pl.ds(0, CZ)],
                            out_hbm.at[pl.ds(cid * NPAD + r0, CZ)])
            return c
        lax.fori_loop(0, RPS // CZ, wbody, 0)

    return k(src, dst, table)


def _vec_pass(src, dst, table):
    C = 400
    CZ = 400
    mesh = plsc.VectorSubcoreMesh(core_axis_name="c", subcore_axis_name="s")

    @functools.partial(
        pl.kernel,
        out_type=jax.ShapeDtypeStruct((NCORE * NPAD, 16), FN),
        mesh=mesh,
        compiler_params=pltpu.CompilerParams(use_tc_tiling_on_sc=False),
        scratch_types=[
            pltpu.VMEM_SHARED((NPAD, 16), FN),
            pltpu.VMEM((C,), jnp.int32),
            pltpu.VMEM((C,), jnp.int32),
            pltpu.VMEM((C,), jnp.int32),
            pltpu.VMEM((C,), jnp.int32),
            pltpu.VMEM((C,), jnp.int32),
            pltpu.VMEM((C,), jnp.int32),
            pltpu.VMEM((C,), jnp.int32),
            pltpu.VMEM((C,), jnp.int32),
            pltpu.VMEM((C, 16), FN),
            pltpu.VMEM((C, 16), FN),
            pltpu.SemaphoreType.DMA,
            pltpu.SemaphoreType.DMA,
            pltpu.SemaphoreType.DMA,
            pltpu.SemaphoreType.DMA,
            pltpu.SemaphoreType.DMA,
            pltpu.SemaphoreType.DMA,
        ],
    )
    def k(src_hbm, dst_hbm, tab_hbm, out_hbm, acc, idxsA0, idxdA0, idxsA1,
          idxdA1, idxsB0, idxdB0, idxsB1, idxdB1, vals0, vals1,
          g0, g1, w0, w1, pA, pB):
        cid = lax.axis_index("c")
        sid = lax.axis_index("s")
        _zero_rows_2d(vals0, CZ)

        def zbody(kk, c):
            pltpu.sync_copy(vals0.at[pl.ds(0, CZ)],
                            acc.at[pl.ds(sid * RPS + kk * CZ, CZ)])
            return c
        lax.fori_loop(0, RPS // CZ, zbody, 0)
        plsc.subcore_barrier()

        base0 = (cid * NSUB + sid) * EPW
        Q = EPW // C // 4

        def pre(bx, by, sA, dA, sB, dB, sem):
            pltpu.async_copy(src_hbm.at[pl.ds(bx, C)], sA, sem)
            pltpu.async_copy(dst_hbm.at[pl.ds(bx, C)], dA, sem)
            pltpu.async_copy(src_hbm.at[pl.ds(by, C)], sB, sem)
            pltpu.async_copy(dst_hbm.at[pl.ds(by, C)], dB, sem)

        def drain(bx, by, sA, dA, sB, dB, sem):
            pltpu.make_async_copy(src_hbm.at[pl.ds(bx, C)], sA, sem).wait()
            pltpu.make_async_copy(dst_hbm.at[pl.ds(bx, C)], dA, sem).wait()
            pltpu.make_async_copy(src_hbm.at[pl.ds(by, C)], sB, sem).wait()
            pltpu.make_async_copy(dst_hbm.at[pl.ds(by, C)], dB, sem).wait()

        pre(base0, base0 + C, idxsA0, idxdA0, idxsA1, idxdA1, pA)

        def ebody(q, c):
            b0 = pl.multiple_of(base0 + (4 * q) * C, 8)
            b1 = pl.multiple_of(base0 + (4 * q + 1) * C, 8)
            b2 = pl.multiple_of(base0 + (4 * q + 2) * C, 8)
            b3 = pl.multiple_of(base0 + (4 * q + 3) * C, 8)
            nxt = jnp.where(q + 1 < Q, base0 + (4 * q + 4) * C, base0)
            nxt = pl.multiple_of(nxt, 8)
            drain(b0, b1, idxsA0, idxdA0, idxsA1, idxdA1, pA)
            ga = pltpu.async_copy(tab_hbm.at[idxsA0], vals0, g0)
            gb = pltpu.async_copy(tab_hbm.at[idxsA1], vals1, g1)
            pre(b2, b3, idxsB0, idxdB0, idxsB1, idxdB1, pB)
            ga.wait()
            sa = pltpu.async_copy(vals0, acc.at[idxdA0], w0, add=True)
            gb.wait()
            sb = pltpu.async_copy(vals1, acc.at[idxdA1], w1, add=True)
            sa.wait()
            sb.wait()
            drain(b2, b3, idxsB0, idxdB0, idxsB1, idxdB1, pB)
            ga2 = pltpu.async_copy(tab_hbm.at[idxsB0], vals0, g0)
            gb2 = pltpu.async_copy(tab_hbm.at[idxsB1], vals1, g1)
            pre(nxt, nxt + C, idxsA0, idxdA0, idxsA1, idxdA1, pA)
            ga2.wait()
            sa2 = pltpu.async_copy(vals0, acc.at[idxdB0], w0, add=True)
            gb2.wait()
            sb2 = pltpu.async_copy(vals1, acc.at[idxdB1], w1, add=True)
            sa2.wait()
            sb2.wait()
            return c
        lax.fori_loop(0, Q, ebody, 0)
        drain(base0, base0 + C, idxsA0, idxdA0, idxsA1, idxdA1, pA)
        plsc.subcore_barrier()

        def wbody(kk, c):
            r0 = sid * RPS + kk * CZ
            pltpu.sync_copy(acc.at[pl.ds(r0, CZ)], vals0.at[pl.ds(0, CZ)])
            pltpu.sync_copy(vals0.at[pl.ds(0, CZ)],
                            out_hbm.at[pl.ds(cid * NPAD + r0, CZ)])
            return c
        lax.fori_loop(0, RPS // CZ, wbody, 0)

    return k(src, dst, table)


def _tc_norm(degp2d, off2d):
    def body(dp_ref, off_ref, o_ref):
        d = dp_ref[0] + dp_ref[1] + off_ref[0, 0]
        o_ref[...] = lax.rsqrt(jnp.maximum(d, 1.0))
    return pl.pallas_call(
        body, out_shape=jax.ShapeDtypeStruct((ROWS2D, 128), FN),
    )(degp2d, off2d)


def _tc_a1(p1p2d, norm2d):
    def body(pp_ref, n_ref, a1_ref, t1_ref):
        nv = n_ref[...]
        a1 = nv * (pp_ref[0] + pp_ref[1])
        a1_ref[...] = a1
        t1_ref[...] = nv * a1
    return pl.pallas_call(
        body,
        out_shape=[jax.ShapeDtypeStruct((ROWS2D, 128), FN),
                   jax.ShapeDtypeStruct((ROWS2D, 128), FN)],
    )(p1p2d, norm2d)


def _tc_a2_u(p2p3, norm2d, a12d, p2pc, a1c, normc, c0, c1, c2, gvec, bvec):
    BR = 1024
    G = NPAD // BR

    def body(pf_ref, n2_ref, a1f_ref, pc_ref, a1c_ref, nc_ref, c0_ref,
             c1_ref, c2_ref, g_ref, b_ref, o_ref, st_ref):
        i = pl.program_id(0)

        @pl.when(i == 0)
        def _():
            a1f = a1f_ref[...]
            a2f = n2_ref[...] * (pf_ref[0] + pf_ref[1])
            inv = jnp.float32(1.0 / N)
            m1 = jnp.sum(a1f) * inv
            m2 = jnp.sum(a2f) * inv
            d1 = a1f - m1
            d2 = a2f - m2
            pcc = jnp.float32(PADC)
            st_ref[0] = m1
            st_ref[1] = m2
            st_ref[2] = (jnp.sum(d1 * d1) - pcc * m1 * m1) * inv
            st_ref[3] = (jnp.sum(d2 * d2) - pcc * m2 * m2) * inv
            st_ref[4] = (jnp.sum(d1 * d2) - pcc * m1 * m2) * inv

        m1 = st_ref[0]
        m2 = st_ref[1]
        c0v = c0_ref[...]
        c1v = c1_ref[...]
        c2v = c2_ref[...]
        var = (c1v * c1v * st_ref[2] + 2.0 * c1v * c2v * st_ref[4]
               + c2v * c2v * st_ref[3])
        sdev = jnp.sqrt(var + 1e-5)
        gs = g_ref[...] / sdev
        mu = c0v + m1 * c1v + m2 * c2v
        alpha = (c0v - mu) * gs + b_ref[...]
        beta = c1v * gs
        gamma = c2v * gs
        a2b = nc_ref[...] * (pc_ref[0] + pc_ref[1])
        u = jnp.maximum(alpha + a1c_ref[...] * beta + a2b * gamma, 0.0)
        o_ref[...] = nc_ref[...] * u

    full = lambda *_: tuple(0 for _ in _)
    return pl.pallas_call(
        body,
        grid=(G,),
        in_specs=[
            pl.BlockSpec((2, ROWS2D, 128), lambda i: (0, 0, 0)),
            pl.BlockSpec((ROWS2D, 128), lambda i: (0, 0)),
            pl.BlockSpec((ROWS2D, 128), lambda i: (0, 0)),
            pl.BlockSpec((2, BR, 1), lambda i: (0, i, 0)),
            pl.BlockSpec((BR, 1), lambda i: (i, 0)),
            pl.BlockSpec((BR, 1), lambda i: (i, 0)),
            pl.BlockSpec((1, 16), lambda i: (0, 0)),
            pl.BlockSpec((1, 16), lambda i: (0, 0)),
            pl.BlockSpec((1, 16), lambda i: (0, 0)),
            pl.BlockSpec((1, 16), lambda i: (0, 0)),
            pl.BlockSpec((1, 16), lambda i: (0, 0)),
        ],
        out_specs=pl.BlockSpec((BR, 16), lambda i: (i, 0)),
        out_shape=jax.ShapeDtypeStruct((NPAD, 16), FN),
        scratch_shapes=[pltpu.SMEM((8,), FN)],
    )(p2p3, norm2d, a12d, p2pc, a1c, normc, c0, c1, c2, gvec, bvec)


def _tc_scale_v(vp, normc):
    BR = 1024
    G = NPAD // BR

    def body(vp_ref, n_ref, o_ref):
        nv = n_ref[...]
        o_ref[...] = (nv * nv) * (vp_ref[0] + vp_ref[1])

    return pl.pallas_call(
        body,
        grid=(G,),
        in_specs=[pl.BlockSpec((2, BR, 16), lambda i: (0, i, 0)),
                  pl.BlockSpec((BR, 1), lambda i: (i, 0))],
        out_specs=pl.BlockSpec((BR, 16), lambda i: (i, 0)),
        out_shape=jax.ShapeDtypeStruct((NPAD, 16), FN),
    )(vp, normc)


_BRH = 1024
_GH = NPAD // _BRH


def _rowmask(i):
    rows = lax.broadcasted_iota(jnp.int32, (_BRH, 1), 0) + i * _BRH
    return (rows < N).astype(FN)


def _tc_head_y(ut, vt1, v2p, normc, p0, p1m, p2m):
    def body(ut_ref, vt1_ref, v2p_ref, n_ref, p0_ref, p1_ref, p2_ref,
             y_ref, ms_ref):
        i = pl.program_id(0)
        nv = n_ref[...]
        u = ut_ref[...] / nv
        v1 = vt1_ref[...] / nv
        v2 = nv * (v2p_ref[0] + v2p_ref[1])
        dn = (((1,), (0,)), ((), ()))
        y = (lax.dot_general(u, p0_ref[...], dn, preferred_element_type=FN)
             + lax.dot_general(v1, p1_ref[...], dn, preferred_element_type=FN)
             + lax.dot_general(v2, p2_ref[...], dn, preferred_element_type=FN))
        y_ref[...] = y

        @pl.when(i == 0)
        def _():
            ms_ref[...] = jnp.zeros((1, 16), FN)

        ms_ref[...] += jnp.sum(y * _rowmask(i), axis=0, keepdims=True)

    blk = pl.BlockSpec((_BRH, 16), lambda i: (i, 0))
    w16 = pl.BlockSpec((16, 16), lambda i: (0, 0))
    return pl.pallas_call(
        body,
        grid=(_GH,),
        in_specs=[blk, blk, pl.BlockSpec((2, _BRH, 16), lambda i: (0, i, 0)),
                  pl.BlockSpec((_BRH, 1), lambda i: (i, 0)), w16, w16, w16],
        out_specs=[blk, pl.BlockSpec((1, 16), lambda i: (0, 0))],
        out_shape=[jax.ShapeDtypeStruct((NPAD, 16), FN),
                   jax.ShapeDtypeStruct((1, 16), FN)],
    )(ut, vt1, v2p, normc, p0, p1m, p2m)


def _tc_head_var(y, msum):
    def body(y_ref, m_ref, vs_ref):
        i = pl.program_id(0)
        m = m_ref[...] * jnp.float32(1.0 / N)
        yc = (y_ref[...] - m) * _rowmask(i)

        @pl.when(i == 0)
        def _():
            vs_ref[...] = jnp.zeros((1, 16), FN)

        vs_ref[...] += jnp.sum(yc * yc, axis=0, keepdims=True)

    return pl.pallas_call(
        body,
        grid=(_GH,),
        in_specs=[pl.BlockSpec((_BRH, 16), lambda i: (i, 0)),
                  pl.BlockSpec((1, 16), lambda i: (0, 0))],
        out_specs=pl.BlockSpec((1, 16), lambda i: (0, 0)),
        out_shape=jax.ShapeDtypeStruct((1, 16), FN),
    )(y, msum)


def _tc_head_relu(y, msum, vsum, gh, bh):
    def body(y_ref, m_ref, v_ref, g_ref, b_ref, ss_ref):
        i = pl.program_id(0)
        inv = jnp.float32(1.0 / N)
        m = m_ref[...] * inv
        v = v_ref[...] * inv
        h = jnp.maximum(
            (y_ref[...] - m) / jnp.sqrt(v + 1e-5) * g_ref[...] + b_ref[...],
            0.0) * _rowmask(i)

        @pl.when(i == 0)
        def _():
            ss_ref[...] = jnp.zeros((1, 16), FN)

        ss_ref[...] += jnp.sum(h, axis=0, keepdims=True)

    r16 = pl.BlockSpec((1, 16), lambda i: (0, 0))
    return pl.pallas_call(
        body,
        grid=(_GH,),
        in_specs=[pl.BlockSpec((_BRH, 16), lambda i: (i, 0)), r16, r16,
                  r16, r16],
        out_specs=pl.BlockSpec((1, 16), lambda i: (0, 0)),
        out_shape=jax.ShapeDtypeStruct((1, 16), FN),
    )(y, msum, vsum, gh, bh)


def kernel(edge_index, num_nodes, h_target_local, h_target_global, W_init,
           b_init, W_gate_l, b_gate_l, W_tag_l1, g_bn_l1, b_bn_l1, W_tag_l2,
           g_bn_l2, b_bn_l2, W_dense_l, b_dense_l, W_ginit, b_ginit, W_gate_g,
           b_gate_g, W_tag_g1, g_bn_g1, b_bn_g1, W_tag_g2, g_bn_g2, b_bn_g2,
           W_dense_g, b_dense_g, W_dec1, b_dec1, W_dec2, b_dec2, W_dec3,
           b_dec3):
    src = edge_index[0]
    dst = edge_index[1]

    degp = _deg_pass(dst)
    off = (jnp.asarray(num_nodes) - N).astype(FN)
    norm2d = _tc_norm(degp.reshape(NCORE, ROWS2D, 128),
                      jnp.full((8, 128), off, FN))
    p1p = _scalar_pass(src, dst, norm2d.reshape(NPAD))
    a12d, t12d = _tc_a1(p1p.reshape(NCORE, ROWS2D, 128), norm2d)
    p2p = _scalar_pass(src, dst, t12d.reshape(NPAD))

    def cvecs(x0, Wg, bg, Wt):
        h0 = jax.nn.sigmoid(x0 @ Wg + bg)
        Cc = h0.shape[0]
        return h0 @ Wt[:Cc], h0 @ Wt[Cc:2 * Cc], h0 @ Wt[2 * Cc:]

    xl0 = jnp.sum(W_init, axis=0) + b_init
    xg0 = jnp.sum(W_ginit, axis=0) + b_ginit
    c0l, c1l, c2l = cvecs(xl0, W_gate_l, b_gate_l, W_tag_l1)
    c0g, c1g, c2g = cvecs(xg0, W_gate_g, b_gate_g, W_tag_g1)
    c0 = jnp.concatenate([c0l, c0g]).reshape(1, 16)
    c1 = jnp.concatenate([c1l, c1g]).reshape(1, 16)
    c2 = jnp.concatenate([c2l, c2g]).reshape(1, 16)
    g1v = jnp.concatenate([g_bn_l1, g_bn_g1]).reshape(1, 16)
    b1v = jnp.concatenate([b_bn_l1, b_bn_g1]).reshape(1, 16)

    a1c = a12d.reshape(NPAD, 1)
    normc = norm2d.reshape(NPAD, 1)
    p2p3 = p2p.reshape(NCORE, ROWS2D, 128)
    p2pc = p2p.reshape(NCORE, NPAD, 1)
    ut = _tc_a2_u(p2p3, norm2d, a12d, p2pc, a1c, normc,
                  c0, c1, c2, g1v, b1v)
    v1p = _vec_pass(src, dst, ut).reshape(NCORE, NPAD, 16)
    vt1 = _tc_scale_v(v1p, normc)
    v2p = _vec_pass(src, dst, vt1).reshape(NCORE, NPAD, 16)

    def bd(wl, wg):
        z = jnp.zeros((16, 16), FN)
        z = z.at[:7, :7].set(wl)
        return z.at[7:, 7:].set(wg)

    p0 = bd(W_tag_l2[0:7], W_tag_g2[0:9])
    p1m = bd(W_tag_l2[7:14], W_tag_g2[9:18])
    p2m = bd(W_tag_l2[14:21], W_tag_g2[18:27])
    gh = jnp.concatenate([g_bn_l2, g_bn_g2]).reshape(1, 16)
    bh = jnp.concatenate([b_bn_l2, b_bn_g2]).reshape(1, 16)
    y, msum = _tc_head_y(ut, vt1, v2p, normc, p0, p1m, p2m)
    vsum = _tc_head_var(y, msum)
    s = _tc_head_relu(y, msum, vsum, gh, bh) * (1.0 / N)

    hg_l = s[0, :7] @ W_dense_l + b_dense_l
    hg_g = s[0, 7:] @ W_dense_g + b_dense_g

    def _cos(a, b):
        return jnp.dot(a, b) / jnp.maximum(
            jnp.linalg.norm(a) * jnp.linalg.norm(b), 1e-6)

    def _dist(a, b):
        return jnp.linalg.norm(a - b + 1e-6)

    feats = jnp.concatenate([
        jnp.stack([_cos(hg_l, h_target_local), _dist(hg_l, h_target_local),
                   _cos(hg_g, h_target_global), _dist(hg_g, h_target_global)]),
        hg_l, h_target_local])[None, :]
    h = jax.nn.relu(feats @ W_dec1 + b_dec1)
    h = jax.nn.relu(h @ W_dec2 + b_dec2)
    return h @ W_dec3 + b_dec3


# speedup vs baseline: 1.0935x; 1.0935x over previous
import functools

import jax
import jax.numpy as jnp
from jax import lax
from jax.experimental import pallas as pl
from jax.experimental.pallas import tpu as pltpu
from jax.experimental.pallas import tpu_sc as plsc

N = 100000
E = 6400000
NPAD = 102400
ROWS2D = 800
PADC = NPAD - N
NCORE = 2
NSUB = 16
NW = NCORE * NSUB
EPW = E // NW
RPS = NPAD // NSUB
FN = jnp.float32


def _fill_1d(ref, n, val):
    def body(i, c):
        ref[pl.ds(i * 16, 16)] = jnp.full((16,), val, FN)
        return c
    lax.fori_loop(0, n // 16, body, 0)


def _zero_rows_2d(ref, n):
    def body(i, c):
        ref[i, :] = jnp.zeros((16,), FN)
        return c
    lax.fori_loop(0, n, body, 0)


def _deg_pass(dst):
    C = 10000
    CZ = 800
    mesh = plsc.VectorSubcoreMesh(core_axis_name="c", subcore_axis_name="s")

    @functools.partial(
        pl.kernel,
        out_type=jax.ShapeDtypeStruct((NCORE * NPAD,), FN),
        mesh=mesh,
        scratch_types=[
            pltpu.VMEM_SHARED((NPAD,), FN),
            pltpu.VMEM((C,), jnp.int32),
            pltpu.VMEM((C,), jnp.int32),
            pltpu.VMEM((C,), FN),
            pltpu.SemaphoreType.DMA,
            pltpu.SemaphoreType.DMA,
        ],
    )
    def k(dst_hbm, out_hbm, acc, idx0, idx1, ones, s0, s1):
        cid = lax.axis_index("c")
        sid = lax.axis_index("s")
        _fill_1d(ones, C, 0.0)

        def zbody(kk, c):
            pltpu.sync_copy(ones.at[pl.ds(0, CZ)],
                            acc.at[pl.ds(sid * RPS + kk * CZ, CZ)])
            return c
        lax.fori_loop(0, RPS // CZ, zbody, 0)
        _fill_1d(ones, C, 1.0)
        plsc.subcore_barrier()

        base0 = (cid * NSUB + sid) * EPW

        def ebody(t, c):
            ba = pl.multiple_of(base0 + (2 * t) * C, 8)
            bb = pl.multiple_of(base0 + (2 * t + 1) * C, 8)
            pltpu.sync_copy(dst_hbm.at[pl.ds(ba, C)], idx0)
            sa = pltpu.async_copy(ones, acc.at[idx0], s0, add=True)
            pltpu.sync_copy(dst_hbm.at[pl.ds(bb, C)], idx1)
            sb = pltpu.async_copy(ones, acc.at[idx1], s1, add=True)
            sa.wait()
            sb.wait()
            return c
        lax.fori_loop(0, EPW // C // 2, ebody, 0)
        plsc.subcore_barrier()

        def wbody(kk, c):
            r0 = sid * RPS + kk * CZ
            pltpu.sync_copy(acc.at[pl.ds(r0, CZ)], ones.at[pl.ds(0, CZ)])
            pltpu.sync_copy(ones.at[pl.ds(0, CZ)],
                            out_hbm.at[pl.ds(cid * NPAD + r0, CZ)])
            return c
        lax.fori_loop(0, RPS // CZ, wbody, 0)

    return k(dst)


def _scalar_pass(src, dst, table):
    C = 10000
    CZ = 800
    ST = 1600
    mesh = plsc.VectorSubcoreMesh(core_axis_name="c", subcore_axis_name="s")

    @functools.partial(
        pl.kernel,
        out_type=jax.ShapeDtypeStruct((NCORE * NPAD,), FN),
        mesh=mesh,
        scratch_types=[
            pltpu.VMEM_SHARED((NPAD,), FN),
            pltpu.VMEM_SHARED((NPAD,), FN),
            pltpu.VMEM((C,), jnp.int32),
            pltpu.VMEM((C,), jnp.int32),
            pltpu.VMEM((C,), jnp.int32),
            pltpu.VMEM((C,), jnp.int32),
            pltpu.VMEM((C,), jnp.int32),
            pltpu.VMEM((C,), jnp.int32),
            pltpu.VMEM((C,), jnp.int32),
            pltpu.VMEM((C,), jnp.int32),
            pltpu.VMEM((C,), FN),
            pltpu.VMEM((C,), FN),
            pltpu.SemaphoreType.DMA,
            pltpu.SemaphoreType.DMA,
            pltpu.SemaphoreType.DMA,
            pltpu.SemaphoreType.DMA,
            pltpu.SemaphoreType.DMA,
            pltpu.SemaphoreType.DMA,
        ],
    )
    def k(src_hbm, dst_hbm, tab_hbm, out_hbm, acc, tab, idxsA0, idxdA0,
          idxsA1, idxdA1, idxsB0, idxdB0, idxsB1, idxdB1, vals0, vals1,
          g0, g1, w0, w1, pA, pB):
        cid = lax.axis_index("c")
        sid = lax.axis_index("s")
        _fill_1d(vals0, C, 0.0)

        def zbody(kk, c):
            pltpu.sync_copy(vals0.at[pl.ds(0, CZ)],
                            acc.at[pl.ds(sid * RPS + kk * CZ, CZ)])
            return c
        lax.fori_loop(0, RPS // CZ, zbody, 0)

        def stage(q, c):
            r0 = sid * RPS + q * ST
            pltpu.sync_copy(tab_hbm.at[pl.ds(r0, ST)], vals1.at[pl.ds(0, ST)])
            pltpu.sync_copy(vals1.at[pl.ds(0, ST)], tab.at[pl.ds(r0, ST)])
            return c
        lax.fori_loop(0, RPS // ST, stage, 0)
        plsc.subcore_barrier()

        base0 = (cid * NSUB + sid) * EPW
        Q = EPW // C // 4

        def pre(bx, by, sA, dA, sB, dB, sem):
            pltpu.async_copy(src_hbm.at[pl.ds(bx, C)], sA, sem)
            pltpu.async_copy(dst_hbm.at[pl.ds(bx, C)], dA, sem)
            pltpu.async_copy(src_hbm.at[pl.ds(by, C)], sB, sem)
            pltpu.async_copy(dst_hbm.at[pl.ds(by, C)], dB, sem)

        def drain(bx, by, sA, dA, sB, dB, sem):
            pltpu.make_async_copy(src_hbm.at[pl.ds(bx, C)], sA, sem).wait()
            pltpu.make_async_copy(dst_hbm.at[pl.ds(bx, C)], dA, sem).wait()
            pltpu.make_async_copy(src_hbm.at[pl.ds(by, C)], sB, sem).wait()
            pltpu.make_async_copy(dst_hbm.at[pl.ds(by, C)], dB, sem).wait()

        pre(base0, base0 + C, idxsA0, idxdA0, idxsA1, idxdA1, pA)

        def ebody(q, c):
            b0 = pl.multiple_of(base0 + (4 * q) * C, 8)
            b1 = pl.multiple_of(base0 + (4 * q + 1) * C, 8)
            b2 = pl.multiple_of(base0 + (4 * q + 2) * C, 8)
            b3 = pl.multiple_of(base0 + (4 * q + 3) * C, 8)
            nxt = jnp.where(q + 1 < Q, base0 + (4 * q + 4) * C, base0)
            nxt = pl.multiple_of(nxt, 8)
            drain(b0, b1, idxsA0, idxdA0, idxsA1, idxdA1, pA)
            ga = pltpu.async_copy(tab.at[idxsA0], vals0, g0)
            gb = pltpu.async_copy(tab.at[idxsA1], vals1, g1)
            pre(b2, b3, idxsB0, idxdB0, idxsB1, idxdB1, pB)
            ga.wait()
            sa = pltpu.async_copy(vals0, acc.at[idxdA0], w0, add=True)
            gb.wait()
            sb = pltpu.async_copy(vals1, acc.at[idxdA1], w1, add=True)
            sa.wait()
            sb.wait()
            drain(b2, b3, idxsB0, idxdB0, idxsB1, idxdB1, pB)
            ga2 = pltpu.async_copy(tab.at[idxsB0], vals0, g0)
            gb2 = pltpu.async_copy(tab.at[idxsB1], vals1, g1)
            pre(nxt, nxt + C, idxsA0, idxdA0, idxsA1, idxdA1, pA)
            ga2.wait()
            sa2 = pltpu.async_copy(vals0, acc.at[idxdB0], w0, add=True)
            gb2.wait()
            sb2 = pltpu.async_copy(vals1, acc.at[idxdB1], w1, add=True)
            sa2.wait()
            sb2.wait()
            return c
        lax.fori_loop(0, Q, ebody, 0)
        drain(base0, base0 + C, idxsA0, idxdA0, idxsA1, idxdA1, pA)
        plsc.subcore_barrier()

        def wbody(kk, c):
            r0 = sid * RPS + kk * CZ
            pltpu.sync_copy(acc.at[pl.ds(r0, CZ)], vals0.at[pl.ds(0, CZ)])
            pltpu.sync_copy(vals0.at[pl.ds(0, CZ)],
                            out_hbm.at[pl.ds(cid * NPAD + r0, CZ)])
            return c
        lax.fori_loop(0, RPS // CZ, wbody, 0)

    return k(src, dst, table)


def _vec_pass(src, dst, table):
    C = 400
    CZ = 400
    mesh = plsc.VectorSubcoreMesh(core_axis_name="c", subcore_axis_name="s")

    @functools.partial(
        pl.kernel,
        out_type=jax.ShapeDtypeStruct((NCORE * NPAD, 16), FN),
        mesh=mesh,
        compiler_params=pltpu.CompilerParams(use_tc_tiling_on_sc=False),
        scratch_types=[
            pltpu.VMEM_SHARED((NPAD, 16), FN),
            pltpu.VMEM((C,), jnp.int32),
            pltpu.VMEM((C,), jnp.int32),
            pltpu.VMEM((C,), jnp.int32),
            pltpu.VMEM((C,), jnp.int32),
            pltpu.VMEM((C,), jnp.int32),
            pltpu.VMEM((C,), jnp.int32),
            pltpu.VMEM((C,), jnp.int32),
            pltpu.VMEM((C,), jnp.int32),
            pltpu.VMEM((C, 16), FN),
            pltpu.VMEM((C, 16), FN),
            pltpu.SemaphoreType.DMA,
            pltpu.SemaphoreType.DMA,
            pltpu.SemaphoreType.DMA,
            pltpu.SemaphoreType.DMA,
            pltpu.SemaphoreType.DMA,
            pltpu.SemaphoreType.DMA,
        ],
    )
    def k(src_hbm, dst_hbm, tab_hbm, out_hbm, acc, idxsA0, idxdA0, idxsA1,
          idxdA1, idxsB0, idxdB0, idxsB1, idxdB1, vals0, vals1,
          g0, g1, w0, w1, pA, pB):
        cid = lax.axis_index("c")
        sid = lax.axis_index("s")
        _zero_rows_2d(vals0, CZ)

        def zbody(kk, c):
            pltpu.sync_copy(vals0.at[pl.ds(0, CZ)],
                            acc.at[pl.ds(sid * RPS + kk * CZ, CZ)])
            return c
        lax.fori_loop(0, RPS // CZ, zbody, 0)
        plsc.subcore_barrier()

        base0 = (cid * NSUB + sid) * EPW
        Q = EPW // C // 4

        def pre(bx, by, sA, dA, sB, dB, sem):
            pltpu.async_copy(src_hbm.at[pl.ds(bx, C)], sA, sem)
            pltpu.async_copy(dst_hbm.at[pl.ds(bx, C)], dA, sem)
            pltpu.async_copy(src_hbm.at[pl.ds(by, C)], sB, sem)
            pltpu.async_copy(dst_hbm.at[pl.ds(by, C)], dB, sem)

        def drain(bx, by, sA, dA, sB, dB, sem):
            pltpu.make_async_copy(src_hbm.at[pl.ds(bx, C)], sA, sem).wait()
            pltpu.make_async_copy(dst_hbm.at[pl.ds(bx, C)], dA, sem).wait()
            pltpu.make_async_copy(src_hbm.at[pl.ds(by, C)], sB, sem).wait()
            pltpu.make_async_copy(dst_hbm.at[pl.ds(by, C)], dB, sem).wait()

        pre(base0, base0 + C, idxsA0, idxdA0, idxsA1, idxdA1, pA)

        def ebody(q, c):
            b0 = pl.multiple_of(base0 + (4 * q) * C, 8)
            b1 = pl.multiple_of(base0 + (4 * q + 1) * C, 8)
            b2 = pl.multiple_of(base0 + (4 * q + 2) * C, 8)
            b3 = pl.multiple_of(base0 + (4 * q + 3) * C, 8)
            nxt = jnp.where(q + 1 < Q, base0 + (4 * q + 4) * C, base0)
            nxt = pl.multiple_of(nxt, 8)
            drain(b0, b1, idxsA0, idxdA0, idxsA1, idxdA1, pA)
            ga = pltpu.async_copy(tab_hbm.at[idxsA0], vals0, g0)
            gb = pltpu.async_copy(tab_hbm.at[idxsA1], vals1, g1)
            pre(b2, b3, idxsB0, idxdB0, idxsB1, idxdB1, pB)
            ga.wait()
            sa = pltpu.async_copy(vals0, acc.at[idxdA0], w0, add=True)
            gb.wait()
            sb = pltpu.async_copy(vals1, acc.at[idxdA1], w1, add=True)
            sa.wait()
            sb.wait()
            drain(b2, b3, idxsB0, idxdB0, idxsB1, idxdB1, pB)
            ga2 = pltpu.async_copy(tab_hbm.at[idxsB0], vals0, g0)
            gb2 = pltpu.async_copy(tab_hbm.at[idxsB1], vals1, g1)
            pre(nxt, nxt + C, idxsA0, idxdA0, idxsA1, idxdA1, pA)
            ga2.wait()
            sa2 = pltpu.async_copy(vals0, acc.at[idxdB0], w0, add=True)
            gb2.wait()
            sb2 = pltpu.async_copy(vals1, acc.at[idxdB1], w1, add=True)
            sa2.wait()
            sb2.wait()
            return c
        lax.fori_loop(0, Q, ebody, 0)
        drain(base0, base0 + C, idxsA0, idxdA0, idxsA1, idxdA1, pA)
        plsc.subcore_barrier()

        def wbody(kk, c):
            r0 = sid * RPS + kk * CZ
            pltpu.sync_copy(acc.at[pl.ds(r0, CZ)], vals0.at[pl.ds(0, CZ)])
            pltpu.sync_copy(vals0.at[pl.ds(0, CZ)],
                            out_hbm.at[pl.ds(cid * NPAD + r0, CZ)])
            return c
        lax.fori_loop(0, RPS // CZ, wbody, 0)

    return k(src, dst, table)


def _tc_norm(degp2d, off2d):
    def body(dp_ref, off_ref, o_ref):
        d = dp_ref[0] + dp_ref[1] + off_ref[0, 0]
        o_ref[...] = lax.rsqrt(jnp.maximum(d, 1.0))
    return pl.pallas_call(
        body, out_shape=jax.ShapeDtypeStruct((ROWS2D, 128), FN),
    )(degp2d, off2d)


def _tc_a1(p1p2d, norm2d):
    def body(pp_ref, n_ref, a1_ref, t1_ref):
        nv = n_ref[...]
        a1 = nv * (pp_ref[0] + pp_ref[1])
        a1_ref[...] = a1
        t1_ref[...] = nv * a1
    return pl.pallas_call(
        body,
        out_shape=[jax.ShapeDtypeStruct((ROWS2D, 128), FN),
                   jax.ShapeDtypeStruct((ROWS2D, 128), FN)],
    )(p1p2d, norm2d)


def _tc_a2_stats(p2p2d, norm2d, a12d):
    def body(pp_ref, n_ref, a1_ref, a2_ref, m1_ref, m2_ref, v11_ref,
             v22_ref, v12_ref):
        a1 = a1_ref[...]
        a2 = n_ref[...] * (pp_ref[0] + pp_ref[1])
        a2_ref[...] = a2
        inv = jnp.float32(1.0 / N)
        m1 = jnp.sum(a1) * inv
        m2 = jnp.sum(a2) * inv
        c1 = a1 - m1
        c2 = a2 - m2
        pc = jnp.float32(PADC)
        v11 = (jnp.sum(c1 * c1) - pc * m1 * m1) * inv
        v22 = (jnp.sum(c2 * c2) - pc * m2 * m2) * inv
        v12 = (jnp.sum(c1 * c2) - pc * m1 * m2) * inv
        m1_ref[...] = jnp.full((1, 1), m1, FN)
        m2_ref[...] = jnp.full((1, 1), m2, FN)
        v11_ref[...] = jnp.full((1, 1), v11, FN)
        v22_ref[...] = jnp.full((1, 1), v22, FN)
        v12_ref[...] = jnp.full((1, 1), v12, FN)
    s = jax.ShapeDtypeStruct((1, 1), FN)
    return pl.pallas_call(
        body,
        out_shape=[jax.ShapeDtypeStruct((ROWS2D, 128), FN), s, s, s, s, s],
    )(p2p2d, norm2d, a12d)


def _tc_build_u(a1c, a2c, normc, alpha, beta, gamma):
    BR = 1024
    G = NPAD // BR

    def body(a1_ref, a2_ref, n_ref, al_ref, be_ref, ga_ref, o_ref):
        u = jnp.maximum(
            al_ref[...] + a1_ref[...] * be_ref[...] + a2_ref[...] * ga_ref[...],
            0.0)
        o_ref[...] = n_ref[...] * u

    col = pl.BlockSpec((BR, 1), lambda i: (i, 0))
    row16 = pl.BlockSpec((1, 16), lambda i: (0, 0))
    return pl.pallas_call(
        body,
        grid=(G,),
        in_specs=[col, col, col, row16, row16, row16],
        out_specs=pl.BlockSpec((BR, 16), lambda i: (i, 0)),
        out_shape=jax.ShapeDtypeStruct((NPAD, 16), FN),
    )(a1c, a2c, normc, alpha, beta, gamma)


def _tc_scale_v(vp, normc):
    BR = 1024
    G = NPAD // BR

    def body(vp_ref, n_ref, o_ref):
        nv = n_ref[...]
        o_ref[...] = (nv * nv) * (vp_ref[0] + vp_ref[1])

    return pl.pallas_call(
        body,
        grid=(G,),
        in_specs=[pl.BlockSpec((2, BR, 16), lambda i: (0, i, 0)),
                  pl.BlockSpec((BR, 1), lambda i: (i, 0))],
        out_specs=pl.BlockSpec((BR, 16), lambda i: (i, 0)),
        out_shape=jax.ShapeDtypeStruct((NPAD, 16), FN),
    )(vp, normc)


_BRH = 1024
_GH = NPAD // _BRH


def _rowmask(i):
    rows = lax.broadcasted_iota(jnp.int32, (_BRH, 1), 0) + i * _BRH
    return (rows < N).astype(FN)


def _tc_head_y(ut, vt1, v2p, normc, p0, p1m, p2m):
    def body(ut_ref, vt1_ref, v2p_ref, n_ref, p0_ref, p1_ref, p2_ref,
             y_ref, ms_ref):
        nv = n_ref[...]
        u = ut_ref[...] / nv
        v1 = vt1_ref[...] / nv
        v2 = nv * (v2p_ref[0] + v2p_ref[1])
        dn = (((1,), (0,)), ((), ()))
        y = (lax.dot_general(u, p0_ref[...], dn, preferred_element_type=FN)
             + lax.dot_general(v1, p1_ref[...], dn, preferred_element_type=FN)
             + lax.dot_general(v2, p2_ref[...], dn, preferred_element_type=FN))
        y_ref[...] = y
        ms_ref[0] = jnp.sum(y * _rowmask(pl.program_id(0)), axis=0,
                            keepdims=True)

    blk = pl.BlockSpec((_BRH, 16), lambda i: (i, 0))
    w16 = pl.BlockSpec((16, 16), lambda i: (0, 0))
    return pl.pallas_call(
        body,
        grid=(_GH,),
        in_specs=[blk, blk, pl.BlockSpec((2, _BRH, 16), lambda i: (0, i, 0)),
                  pl.BlockSpec((_BRH, 1), lambda i: (i, 0)), w16, w16, w16],
        out_specs=[blk, pl.BlockSpec((1, 1, 16), lambda i: (i, 0, 0))],
        out_shape=[jax.ShapeDtypeStruct((NPAD, 16), FN),
                   jax.ShapeDtypeStruct((_GH, 1, 16), FN)],
    )(ut, vt1, v2p, normc, p0, p1m, p2m)


def _tc_head_var(y, m):
    def body(y_ref, m_ref, vs_ref):
        yc = (y_ref[...] - m_ref[...]) * _rowmask(pl.program_id(0))
        vs_ref[0] = jnp.sum(yc * yc, axis=0, keepdims=True)

    return pl.pallas_call(
        body,
        grid=(_GH,),
        in_specs=[pl.BlockSpec((_BRH, 16), lambda i: (i, 0)),
                  pl.BlockSpec((1, 16), lambda i: (0, 0))],
        out_specs=pl.BlockSpec((1, 1, 16), lambda i: (i, 0, 0)),
        out_shape=jax.ShapeDtypeStruct((_GH, 1, 16), FN),
    )(y, m)


def _tc_head_relu(y, m, v, gh, bh):
    def body(y_ref, m_ref, v_ref, g_ref, b_ref, ss_ref):
        h = jnp.maximum(
            (y_ref[...] - m_ref[...]) / jnp.sqrt(v_ref[...] + 1e-5)
            * g_ref[...] + b_ref[...], 0.0) * _rowmask(pl.program_id(0))
        ss_ref[0] = jnp.sum(h, axis=0, keepdims=True)

    r16 = pl.BlockSpec((1, 16), lambda i: (0, 0))
    return pl.pallas_call(
        body,
        grid=(_GH,),
        in_specs=[pl.BlockSpec((_BRH, 16), lambda i: (i, 0)), r16, r16,
                  r16, r16],
        out_specs=pl.BlockSpec((1, 1, 16), lambda i: (i, 0, 0)),
        out_shape=jax.ShapeDtypeStruct((_GH, 1, 16), FN),
    )(y, m, v, gh, bh)


def kernel(edge_index, num_nodes, h_target_local, h_target_global, W_init,
           b_init, W_gate_l, b_gate_l, W_tag_l1, g_bn_l1, b_bn_l1, W_tag_l2,
           g_bn_l2, b_bn_l2, W_dense_l, b_dense_l, W_ginit, b_ginit, W_gate_g,
           b_gate_g, W_tag_g1, g_bn_g1, b_bn_g1, W_tag_g2, g_bn_g2, b_bn_g2,
           W_dense_g, b_dense_g, W_dec1, b_dec1, W_dec2, b_dec2, W_dec3,
           b_dec3):
    src = edge_index[0]
    dst = edge_index[1]

    degp = _deg_pass(dst)
    off = (jnp.asarray(num_nodes) - N).astype(FN)
    norm2d = _tc_norm(degp.reshape(NCORE, ROWS2D, 128),
                      jnp.full((8, 128), off, FN))
    p1p = _scalar_pass(src, dst, norm2d.reshape(NPAD))
    a12d, t12d = _tc_a1(p1p.reshape(NCORE, ROWS2D, 128), norm2d)
    p2p = _scalar_pass(src, dst, t12d.reshape(NPAD))
    a22d, m1, m2, v11, v22, v12 = _tc_a2_stats(
        p2p.reshape(NCORE, ROWS2D, 128), norm2d, a12d)
    m1 = m1[0, 0]; m2 = m2[0, 0]
    v11 = v11[0, 0]; v22 = v22[0, 0]; v12 = v12[0, 0]

    def coeffs(x0, Wg, bg, Wt, g, b):
        h0 = jax.nn.sigmoid(x0 @ Wg + bg)
        Cc = h0.shape[0]
        c0 = h0 @ Wt[:Cc]
        c1 = h0 @ Wt[Cc:2 * Cc]
        c2 = h0 @ Wt[2 * Cc:]
        var = c1 * c1 * v11 + 2.0 * c1 * c2 * v12 + c2 * c2 * v22
        sdev = jnp.sqrt(var + 1e-5)
        mu = c0 + m1 * c1 + m2 * c2
        return ((c0 - mu) / sdev * g + b, c1 / sdev * g, c2 / sdev * g)

    xl0 = jnp.sum(W_init, axis=0) + b_init
    xg0 = jnp.sum(W_ginit, axis=0) + b_ginit
    al, be_l, ga_l = coeffs(xl0, W_gate_l, b_gate_l, W_tag_l1, g_bn_l1, b_bn_l1)
    ag, be_g, ga_g = coeffs(xg0, W_gate_g, b_gate_g, W_tag_g1, g_bn_g1, b_bn_g1)
    alpha = jnp.concatenate([al, ag]).reshape(1, 16)
    beta = jnp.concatenate([be_l, be_g]).reshape(1, 16)
    gamma = jnp.concatenate([ga_l, ga_g]).reshape(1, 16)

    a1c = a12d.reshape(NPAD, 1)
    a2c = a22d.reshape(NPAD, 1)
    normc = norm2d.reshape(NPAD, 1)
    ut = _tc_build_u(a1c, a2c, normc, alpha, beta, gamma)
    v1p = _vec_pass(src, dst, ut).reshape(NCORE, NPAD, 16)
    vt1 = _tc_scale_v(v1p, normc)
    v2p = _vec_pass(src, dst, vt1).reshape(NCORE, NPAD, 16)

    def bd(wl, wg):
        z = jnp.zeros((16, 16), FN)
        z = z.at[:7, :7].set(wl)
        return z.at[7:, 7:].set(wg)

    p0 = bd(W_tag_l2[0:7], W_tag_g2[0:9])
    p1m = bd(W_tag_l2[7:14], W_tag_g2[9:18])
    p2m = bd(W_tag_l2[14:21], W_tag_g2[18:27])
    gh = jnp.concatenate([g_bn_l2, g_bn_g2]).reshape(1, 16)
    bh = jnp.concatenate([b_bn_l2, b_bn_g2]).reshape(1, 16)
    y, mpart = _tc_head_y(ut, vt1, v2p, normc, p0, p1m, p2m)
    m = jnp.sum(mpart, axis=0) * (1.0 / N)
    vpart = _tc_head_var(y, m)
    v = jnp.sum(vpart, axis=0) * (1.0 / N)
    spart = _tc_head_relu(y, m, v, gh, bh)
    s = jnp.sum(spart, axis=0) * (1.0 / N)

    hg_l = s[0, :7] @ W_dense_l + b_dense_l
    hg_g = s[0, 7:] @ W_dense_g + b_dense_g

    def _cos(a, b):
        return jnp.dot(a, b) / jnp.maximum(
            jnp.linalg.norm(a) * jnp.linalg.norm(b), 1e-6)

    def _dist(a, b):
        return jnp.linalg.norm(a - b + 1e-6)

    feats = jnp.concatenate([
        jnp.stack([_cos(hg_l, h_target_local), _dist(hg_l, h_target_local),
                   _cos(hg_g, h_target_global), _dist(hg_g, h_target_global)]),
        hg_l, h_target_local])[None, :]
    h = jax.nn.relu(feats @ W_dec1 + b_dec1)
    h = jax.nn.relu(h @ W_dec2 + b_dec2)
    return h @ W_dec3 + b_dec3
